# baseline (speedup 1.0000x reference)
import functools

import jax
import jax.numpy as jnp
from jax import lax
from jax.experimental import pallas as pl
from jax.experimental.pallas import tpu as pltpu, tpu_sc as plsc

B, T_PH, T_W = 16, 2048, 1024
EPS = 1e-05
L = 16
H = T_PH // 2
MAGIC = 12582912.0


def _body(ph_hbm, idx_hbm, wd_hbm, zero_hbm, out_hbm,
          ph_v, idx_v, wd_v, seg_v, pub_w_v, pub_p_v, out_v,
          shr_w, shr_p, sem):
    core = lax.axis_index("c")
    sub = lax.axis_index("s")
    row = core * 8 + sub // 2
    half = sub % 2
    base = half * H

    cp_ph = pltpu.async_copy(ph_hbm.at[row, pl.ds(base, H)], ph_v, sem)
    cp_ix = pltpu.async_copy(idx_hbm.at[row, pl.ds(base, H)], idx_v, sem)
    cp_wd = pltpu.async_copy(wd_hbm.at[row], wd_v, sem)
    cp_z = pltpu.async_copy(zero_hbm, seg_v, sem)
    cp_ph.wait()
    cp_ix.wait()
    cp_wd.wait()
    cp_z.wait()

    def scat_step(i, _):
        idx = idx_v[pl.ds(i * L, L)]
        vals = ph_v[pl.ds(i * L, L)]
        mask = idx > 0
        plsc.addupdate_scatter(seg_v, [jnp.maximum(idx - 1, 0)], vals,
                               mask=mask)
        return 0

    lax.fori_loop(0, H // L, scat_step, 0, unroll=4)

    last_chunk = idx_v[pl.ds(H - L, L)]
    first_chunk = idx_v[pl.ds(0, L)]
    w_edge = jnp.where(half == 0,
                       lax.reduce_max(last_chunk, axes=(0,)),
                       lax.reduce_min(first_chunk, axes=(0,)))
    w_splat = jnp.full((L,), w_edge, jnp.int32)
    part = plsc.load_gather(seg_v, [jnp.maximum(w_splat - 1, 0)],
                            mask=w_splat > 0)
    pub_w_v[...] = w_splat
    pub_p_v[...] = part
    pltpu.sync_copy(pub_w_v, shr_w.at[sub])
    pltpu.sync_copy(pub_p_v, shr_p.at[sub])
    plsc.subcore_barrier()
    pltpu.sync_copy(shr_w.at[sub - 2 * half + 1], pub_w_v)
    pltpu.sync_copy(shr_p.at[sub - 2 * half + 1], pub_p_v)
    w_part = pub_w_v[...]
    p_part = pub_p_v[...]
    lane0 = lax.iota(jnp.int32, L) == 0
    match = (w_part == w_splat) & (w_splat > 0) & lane0
    plsc.addupdate_scatter(seg_v, [jnp.maximum(w_splat - 1, 0)], p_part,
                           mask=match)

    def gath_step(i, _):
        idx = idx_v[pl.ds(i * L, L)]
        vals = ph_v[pl.ds(i * L, L)]
        mask = idx > 0
        gi = jnp.maximum(idx - 1, 0)
        s = plsc.load_gather(seg_v, [gi], mask=mask)
        w = plsc.load_gather(wd_v, [gi], mask=mask)
        a = w / jnp.maximum(s, EPS)
        x = jnp.where(mask, vals * a, 0.0)
        r = (x + MAGIC) - MAGIC
        out_v[pl.ds(i * L, L)] = r.astype(jnp.int32)
        return 0

    lax.fori_loop(0, H // L, gath_step, 0, unroll=4)

    pltpu.sync_copy(out_v, out_hbm.at[row, pl.ds(base, H)])


@jax.jit
def _regulate(ph_dur, ph2word_i32, word_dur):
    mesh = plsc.VectorSubcoreMesh(core_axis_name="c", subcore_axis_name="s")
    f = functools.partial(
        pl.kernel,
        out_type=jax.ShapeDtypeStruct((B, T_PH), jnp.int32),
        mesh=mesh,
        compiler_params=pltpu.CompilerParams(needs_layout_passes=False),
        scratch_types=[
            pltpu.VMEM((H,), jnp.float32),
            pltpu.VMEM((H,), jnp.int32),
            pltpu.VMEM((T_W,), jnp.float32),
            pltpu.VMEM((T_W,), jnp.float32),
            pltpu.VMEM((L,), jnp.int32),
            pltpu.VMEM((L,), jnp.float32),
            pltpu.VMEM((H,), jnp.int32),
            pltpu.VMEM_SHARED((16, L), jnp.int32),
            pltpu.VMEM_SHARED((16, L), jnp.float32),
            pltpu.SemaphoreType.DMA,
        ],
    )(_body)
    zeros = jnp.zeros((T_W,), jnp.float32)
    return f(ph_dur, ph2word_i32, word_dur, zeros)


def kernel(ph_dur, ph2word, word_dur):
    out = _regulate(ph_dur.astype(jnp.float32), ph2word.astype(jnp.int32),
                    word_dur.astype(jnp.float32))
    return out.astype(jnp.int64)
